# Initial kernel scaffold; baseline (speedup 1.0000x reference)
import jax
import jax.numpy as jnp
from jax.experimental import pallas as pl


def kernel(x, edge_index, W_pools, lin_w, lin_b):
    raise NotImplementedError("write your pallas kernel here")


# speedup vs baseline: 3.4988x; 3.4988x over previous
import functools

import jax
import jax.numpy as jnp
from jax import lax
from jax.experimental import pallas as pl
from jax.experimental.pallas import tpu as pltpu
from jax.experimental.pallas import tpu_sc as plsc

N = 10000
D = 128
HD = 64
E = 320000
RANK = 64
ODIM = 32

NC = 2
NS = 16
NW = NC * NS
CHUNK = 128
CHUNKS = 79
EPAD = NW * CHUNKS * CHUNK
NPAD = 10240
RPT = NPAD // NS

_mesh = plsc.VectorSubcoreMesh(core_axis_name="c", subcore_axis_name="s")


@functools.partial(
    pl.kernel,
    out_type=[
        jax.ShapeDtypeStruct((NC, NPAD, HD), jnp.float32),
        jax.ShapeDtypeStruct((NC, NPAD, HD), jnp.float32),
    ],
    mesh=_mesh,
    compiler_params=pltpu.CompilerParams(use_tc_tiling_on_sc=False),
    scratch_types=[
        pltpu.VMEM((CHUNKS, CHUNK), jnp.int32),
        pltpu.VMEM((CHUNKS, CHUNK), jnp.int32),
        pltpu.VMEM((CHUNK, HD), jnp.float32),
        pltpu.VMEM((CHUNK, HD), jnp.float32),
        pltpu.VMEM_SHARED((NPAD, HD), jnp.float32),
        pltpu.SemaphoreType.DMA,
    ],
)
def _sc_agg(h0_hbm, h1_hbm, col_hbm, row_hbm, zero_hbm, out0_hbm, out1_hbm,
            cidx, ridx, rows, zbuf, agg_sh, sem):
    c = lax.axis_index("c")
    s = lax.axis_index("s")
    w = c * NS + s

    pltpu.sync_copy(col_hbm.at[w], cidx)
    pltpu.sync_copy(row_hbm.at[w], ridx)
    pltpu.sync_copy(zero_hbm, zbuf)
    base = s * RPT

    for hf_hbm, out_hbm in ((h0_hbm, out0_hbm), (h1_hbm, out1_hbm)):
        for k in range(RPT // CHUNK):
            pltpu.sync_copy(zbuf, agg_sh.at[pl.ds(base + k * CHUNK, CHUNK)])
        plsc.subcore_barrier()

        def chunk_body(j, carry):
            pltpu.async_copy(hf_hbm.at[cidx.at[j]], rows, sem).wait()
            pltpu.sync_copy(rows, agg_sh.at[ridx.at[j]], add=True)
            return carry

        lax.fori_loop(0, CHUNKS, chunk_body, 0)
        plsc.subcore_barrier()

        pltpu.sync_copy(agg_sh.at[pl.ds(base, RPT)],
                        out_hbm.at[c, pl.ds(base, RPT)])


BLK = 1000


def _block_prod(t):
    a = jnp.concatenate([t, jnp.ones((1024 - BLK, t.shape[1]), t.dtype)], axis=0)
    for half in (512, 256, 128, 64, 32, 16, 8):
        a = a[0:half] * a[half:2 * half]
    return a


def _rows8_prod(a):
    r = a[0:1]
    for i in range(1, 8):
        r = r * a[i:i + 1]
    return r


def _tanh_z(h0, h1, W):
    z = (lax.dot_general(h0, W[0:HD], (((1,), (0,)), ((), ())),
                         precision=lax.Precision.HIGHEST)
         + lax.dot_general(h1, W[HD:D], (((1,), (0,)), ((), ())),
                           precision=lax.Precision.HIGHEST)
         + W[D:D + 1])
    return jnp.tanh(z)


def _tc_mid_body(x0_ref, x1_ref, a0_ref, a1_ref, W_ref,
                 h0_ref, h1_ref, p0_ref, p1_ref):
    W0 = W_ref[0]
    W1 = W_ref[1]

    def blk(i, accs):
        acc0, acc1 = accs
        sl = pl.ds(i * BLK, BLK)
        x0 = x0_ref[sl, :]
        x1 = x1_ref[sl, :]
        h0 = x0 + a0_ref[0, sl, :] + a0_ref[1, sl, :]
        h1 = x1 + a1_ref[0, sl, :] + a1_ref[1, sl, :]
        h0_ref[sl, :] = h0
        h1_ref[sl, :] = h1
        acc0 = acc0 * _block_prod(_tanh_z(x0, x1, W0))
        acc1 = acc1 * _block_prod(_tanh_z(h0, h1, W1))
        return (acc0, acc1)

    ones = jnp.ones((8, RANK), jnp.float32)
    acc0, acc1 = lax.fori_loop(0, N // BLK, blk, (ones, ones))
    p0_ref[...] = _rows8_prod(acc0)
    p1_ref[...] = _rows8_prod(acc1)


def _tc_final_body(h0_ref, h1_ref, a0_ref, a1_ref, W_ref, p0_ref, p1_ref,
                   lw_ref, lb_ref, score_ref):
    W2 = W_ref[2]

    def blk(i, acc):
        sl = pl.ds(i * BLK, BLK)
        g0 = h0_ref[sl, :] + a0_ref[0, sl, :] + a0_ref[1, sl, :]
        g1 = h1_ref[sl, :] + a1_ref[0, sl, :] + a1_ref[1, sl, :]
        return acc * _block_prod(_tanh_z(g0, g1, W2))

    acc = lax.fori_loop(0, N // BLK, blk, jnp.ones((8, RANK), jnp.float32))
    p2 = _rows8_prod(acc)
    pools = (p0_ref[...], p1_ref[...], p2)
    score = jnp.zeros((1, ODIM), jnp.float32)
    for l in range(3):
        score = score + lax.dot_general(
            pools[l], lw_ref[l], (((1,), (1,)), ((), ())),
            precision=lax.Precision.HIGHEST) + lb_ref[l:l + 1, :]
    score_ref[...] = score


_tc_mid = pl.pallas_call(
    _tc_mid_body,
    out_shape=[
        jax.ShapeDtypeStruct((N, HD), jnp.float32),
        jax.ShapeDtypeStruct((N, HD), jnp.float32),
        jax.ShapeDtypeStruct((1, RANK), jnp.float32),
        jax.ShapeDtypeStruct((1, RANK), jnp.float32),
    ],
)

_tc_final = pl.pallas_call(
    _tc_final_body,
    out_shape=jax.ShapeDtypeStruct((1, ODIM), jnp.float32),
)


def kernel(x, edge_index, W_pools, lin_w, lin_b):
    row = edge_index[0]
    col = edge_index[1]
    pad = EPAD - E
    row_p = jnp.concatenate(
        [row, jnp.full((pad,), N, jnp.int32)]).reshape(NW, CHUNKS, CHUNK)
    col_p = jnp.concatenate(
        [col, jnp.zeros((pad,), jnp.int32)]).reshape(NW, CHUNKS, CHUNK)
    zeros = jnp.zeros((CHUNK, HD), jnp.float32)
    x0 = x[:, 0:HD]
    x1 = x[:, HD:D]

    a0, a1 = _sc_agg(x0, x1, col_p, row_p, zeros)
    h0, h1, p0, p1 = _tc_mid(x0, x1, a0, a1, W_pools)
    b0, b1 = _sc_agg(h0, h1, col_p, row_p, zeros)
    return _tc_final(h0, h1, b0, b1, W_pools, p0, p1, lin_w, lin_b)


# speedup vs baseline: 4.3913x; 1.2551x over previous
import functools

import jax
import jax.numpy as jnp
from jax import lax
from jax.experimental import pallas as pl
from jax.experimental.pallas import tpu as pltpu
from jax.experimental.pallas import tpu_sc as plsc

N = 10000
D = 128
HD = 64
E = 320000
RANK = 64
ODIM = 32

NC = 2
NS = 16
NW = NC * NS
CHUNK = 128
CHUNKS = 79
EPAD = NW * CHUNKS * CHUNK
NPAD = 10240
RPT = NPAD // NS

_mesh = plsc.VectorSubcoreMesh(core_axis_name="c", subcore_axis_name="s")


@functools.partial(
    pl.kernel,
    out_type=[
        jax.ShapeDtypeStruct((NC, NPAD, HD), jnp.float32),
        jax.ShapeDtypeStruct((NC, NPAD, HD), jnp.float32),
    ],
    mesh=_mesh,
    compiler_params=pltpu.CompilerParams(use_tc_tiling_on_sc=False),
    scratch_types=[
        pltpu.VMEM((CHUNKS, CHUNK), jnp.int32),
        pltpu.VMEM((CHUNKS, CHUNK), jnp.int32),
        pltpu.VMEM((CHUNK, HD), jnp.float32),
        pltpu.VMEM((CHUNK, HD), jnp.float32),
        pltpu.VMEM((CHUNK, HD), jnp.float32),
        pltpu.VMEM_SHARED((NPAD, HD), jnp.float32),
        pltpu.SemaphoreType.DMA,
        pltpu.SemaphoreType.DMA,
    ],
)
def _sc_agg(h0_hbm, h1_hbm, col_hbm, row_hbm, zero_hbm, out0_hbm, out1_hbm,
            cidx, ridx, rows_a, rows_b, zbuf, agg_sh, sem_a, sem_b):
    c = lax.axis_index("c")
    s = lax.axis_index("s")
    w = c * NS + s

    pltpu.sync_copy(col_hbm.at[w], cidx)
    pltpu.sync_copy(row_hbm.at[w], ridx)
    pltpu.sync_copy(zero_hbm, zbuf)
    base = s * RPT

    for hf_hbm, out_hbm in ((h0_hbm, out0_hbm), (h1_hbm, out1_hbm)):
        for k in range(RPT // CHUNK):
            pltpu.sync_copy(zbuf, agg_sh.at[pl.ds(base + k * CHUNK, CHUNK)])
        plsc.subcore_barrier()

        pltpu.async_copy(hf_hbm.at[cidx.at[0]], rows_a, sem_a)
        pltpu.async_copy(hf_hbm.at[cidx.at[1]], rows_b, sem_b)

        def pair_body(jj, carry):
            j = jj * 2
            pltpu.make_async_copy(hf_hbm.at[cidx.at[j]], rows_a, sem_a).wait()
            pltpu.sync_copy(rows_a, agg_sh.at[ridx.at[j]], add=True)
            pltpu.async_copy(hf_hbm.at[cidx.at[j + 2]], rows_a, sem_a)
            pltpu.make_async_copy(
                hf_hbm.at[cidx.at[j + 1]], rows_b, sem_b).wait()
            pltpu.sync_copy(rows_b, agg_sh.at[ridx.at[j + 1]], add=True)

            @pl.when(j + 3 < CHUNKS)
            def _():
                pltpu.async_copy(hf_hbm.at[cidx.at[j + 3]], rows_b, sem_b)

            return carry

        lax.fori_loop(0, (CHUNKS - 1) // 2, pair_body, 0)
        pltpu.make_async_copy(
            hf_hbm.at[cidx.at[CHUNKS - 1]], rows_a, sem_a).wait()
        pltpu.sync_copy(rows_a, agg_sh.at[ridx.at[CHUNKS - 1]], add=True)
        plsc.subcore_barrier()

        pltpu.sync_copy(agg_sh.at[pl.ds(base, RPT)],
                        out_hbm.at[c, pl.ds(base, RPT)])


BLK = 1000


def _block_prod(t):
    a = jnp.concatenate([t, jnp.ones((1024 - BLK, t.shape[1]), t.dtype)], axis=0)
    for half in (512, 256, 128, 64, 32, 16, 8):
        a = a[0:half] * a[half:2 * half]
    return a


def _rows8_prod(a):
    r = a[0:1]
    for i in range(1, 8):
        r = r * a[i:i + 1]
    return r


def _tanh_z(h0, h1, W):
    z = (lax.dot_general(h0, W[0:HD], (((1,), (0,)), ((), ())),
                         precision=lax.Precision.HIGHEST)
         + lax.dot_general(h1, W[HD:D], (((1,), (0,)), ((), ())),
                           precision=lax.Precision.HIGHEST)
         + W[D:D + 1])
    return jnp.tanh(z)


def _tc_mid_body(x0_ref, x1_ref, a0_ref, a1_ref, W_ref,
                 h0_ref, h1_ref, p0_ref, p1_ref):
    W0 = W_ref[0]
    W1 = W_ref[1]

    def blk(i, accs):
        acc0, acc1 = accs
        sl = pl.ds(i * BLK, BLK)
        x0 = x0_ref[sl, :]
        x1 = x1_ref[sl, :]
        h0 = x0 + a0_ref[0, sl, :] + a0_ref[1, sl, :]
        h1 = x1 + a1_ref[0, sl, :] + a1_ref[1, sl, :]
        h0_ref[sl, :] = h0
        h1_ref[sl, :] = h1
        acc0 = acc0 * _block_prod(_tanh_z(x0, x1, W0))
        acc1 = acc1 * _block_prod(_tanh_z(h0, h1, W1))
        return (acc0, acc1)

    ones = jnp.ones((8, RANK), jnp.float32)
    acc0, acc1 = lax.fori_loop(0, N // BLK, blk, (ones, ones))
    p0_ref[...] = _rows8_prod(acc0)
    p1_ref[...] = _rows8_prod(acc1)


def _tc_final_body(h0_ref, h1_ref, a0_ref, a1_ref, W_ref, p0_ref, p1_ref,
                   lw_ref, lb_ref, score_ref):
    W2 = W_ref[2]

    def blk(i, acc):
        sl = pl.ds(i * BLK, BLK)
        g0 = h0_ref[sl, :] + a0_ref[0, sl, :] + a0_ref[1, sl, :]
        g1 = h1_ref[sl, :] + a1_ref[0, sl, :] + a1_ref[1, sl, :]
        return acc * _block_prod(_tanh_z(g0, g1, W2))

    acc = lax.fori_loop(0, N // BLK, blk, jnp.ones((8, RANK), jnp.float32))
    p2 = _rows8_prod(acc)
    pools = (p0_ref[...], p1_ref[...], p2)
    score = jnp.zeros((1, ODIM), jnp.float32)
    for l in range(3):
        score = score + lax.dot_general(
            pools[l], lw_ref[l], (((1,), (1,)), ((), ())),
            precision=lax.Precision.HIGHEST) + lb_ref[l:l + 1, :]
    score_ref[...] = score


_tc_mid = pl.pallas_call(
    _tc_mid_body,
    out_shape=[
        jax.ShapeDtypeStruct((N, HD), jnp.float32),
        jax.ShapeDtypeStruct((N, HD), jnp.float32),
        jax.ShapeDtypeStruct((1, RANK), jnp.float32),
        jax.ShapeDtypeStruct((1, RANK), jnp.float32),
    ],
)

_tc_final = pl.pallas_call(
    _tc_final_body,
    out_shape=jax.ShapeDtypeStruct((1, ODIM), jnp.float32),
)


def kernel(x, edge_index, W_pools, lin_w, lin_b):
    row = edge_index[0]
    col = edge_index[1]
    pad = EPAD - E
    row_p = jnp.concatenate(
        [row, jnp.full((pad,), N, jnp.int32)]).reshape(NW, CHUNKS, CHUNK)
    col_p = jnp.concatenate(
        [col, jnp.zeros((pad,), jnp.int32)]).reshape(NW, CHUNKS, CHUNK)
    zeros = jnp.zeros((CHUNK, HD), jnp.float32)
    x0 = x[:, 0:HD]
    x1 = x[:, HD:D]

    a0, a1 = _sc_agg(x0, x1, col_p, row_p, zeros)
    h0, h1, p0, p1 = _tc_mid(x0, x1, a0, a1, W_pools)
    b0, b1 = _sc_agg(h0, h1, col_p, row_p, zeros)
    return _tc_final(h0, h1, b0, b1, W_pools, p0, p1, lin_w, lin_b)


# speedup vs baseline: 4.5955x; 1.0465x over previous
import functools

import jax
import jax.numpy as jnp
from jax import lax
from jax.experimental import pallas as pl
from jax.experimental.pallas import tpu as pltpu
from jax.experimental.pallas import tpu_sc as plsc

N = 10000
D = 128
HD = 64
E = 320000
RANK = 64
ODIM = 32

NC = 2
NS = 16
NW = NC * NS
CHUNK = 128
CHUNKS = 79
EPAD = NW * CHUNKS * CHUNK
NPAD = 10240
RPT = NPAD // NS

_mesh = plsc.VectorSubcoreMesh(core_axis_name="c", subcore_axis_name="s")


@functools.partial(
    pl.kernel,
    out_type=[
        jax.ShapeDtypeStruct((NC, NPAD, HD), jnp.float32),
        jax.ShapeDtypeStruct((NC, NPAD, HD), jnp.float32),
    ],
    mesh=_mesh,
    compiler_params=pltpu.CompilerParams(use_tc_tiling_on_sc=False),
    scratch_types=[
        pltpu.VMEM((CHUNKS, CHUNK), jnp.int32),
        pltpu.VMEM((CHUNKS, CHUNK), jnp.int32),
        pltpu.VMEM((CHUNK, HD), jnp.float32),
        pltpu.VMEM((CHUNK, HD), jnp.float32),
        pltpu.VMEM((CHUNK, HD), jnp.float32),
        pltpu.VMEM((CHUNK, HD), jnp.float32),
        pltpu.VMEM_SHARED((NPAD, HD), jnp.float32),
        pltpu.SemaphoreType.DMA,
        pltpu.SemaphoreType.DMA,
        pltpu.SemaphoreType.DMA,
    ],
)
def _sc_agg(h0_hbm, h1_hbm, col_hbm, row_hbm, zero_hbm, out0_hbm, out1_hbm,
            cidx, ridx, rows_a, rows_b, rows_c, zbuf, agg_sh,
            sem_a, sem_b, sem_c):
    c = lax.axis_index("c")
    s = lax.axis_index("s")
    w = c * NS + s

    pltpu.sync_copy(col_hbm.at[w], cidx)
    pltpu.sync_copy(row_hbm.at[w], ridx)
    pltpu.sync_copy(zero_hbm, zbuf)
    base = s * RPT

    for hf_hbm, out_hbm in ((h0_hbm, out0_hbm), (h1_hbm, out1_hbm)):
        for k in range(RPT // CHUNK):
            pltpu.sync_copy(zbuf, agg_sh.at[pl.ds(base + k * CHUNK, CHUNK)])
        plsc.subcore_barrier()

        bufs = ((rows_a, sem_a), (rows_b, sem_b), (rows_c, sem_c))
        for k in range(3):
            pltpu.async_copy(hf_hbm.at[cidx.at[k]], bufs[k][0], bufs[k][1])

        def trio_body(jj, carry):
            j0 = jj * 3
            for k in range(3):
                j = j0 + k
                buf, sem = bufs[k]
                pltpu.make_async_copy(hf_hbm.at[cidx.at[j]], buf, sem).wait()
                pltpu.sync_copy(buf, agg_sh.at[ridx.at[j]], add=True)

                @pl.when(j + 3 < CHUNKS)
                def _():
                    pltpu.async_copy(hf_hbm.at[cidx.at[j + 3]], buf, sem)

            return carry

        lax.fori_loop(0, CHUNKS // 3, trio_body, 0)
        for j in range(CHUNKS - CHUNKS % 3, CHUNKS):
            buf, sem = bufs[j % 3]
            pltpu.make_async_copy(hf_hbm.at[cidx.at[j]], buf, sem).wait()
            pltpu.sync_copy(buf, agg_sh.at[ridx.at[j]], add=True)
        plsc.subcore_barrier()

        pltpu.sync_copy(agg_sh.at[pl.ds(base, RPT)],
                        out_hbm.at[c, pl.ds(base, RPT)])


BLK = 1000


def _block_prod(t):
    a = jnp.concatenate([t, jnp.ones((1024 - BLK, t.shape[1]), t.dtype)], axis=0)
    for half in (512, 256, 128, 64, 32, 16, 8):
        a = a[0:half] * a[half:2 * half]
    return a


def _rows8_prod(a):
    r = a[0:1]
    for i in range(1, 8):
        r = r * a[i:i + 1]
    return r


def _tanh_z(h0, h1, W):
    z = (lax.dot_general(h0, W[0:HD], (((1,), (0,)), ((), ())),
                         precision=lax.Precision.HIGHEST)
         + lax.dot_general(h1, W[HD:D], (((1,), (0,)), ((), ())),
                           precision=lax.Precision.HIGHEST)
         + W[D:D + 1])
    return jnp.tanh(z)


def _tc_pool0_body(x0_ref, x1_ref, W_ref, p0_ref):
    W0 = W_ref[0]

    def blk(i, acc):
        sl = pl.ds(i * BLK, BLK)
        return acc * _block_prod(_tanh_z(x0_ref[sl, :], x1_ref[sl, :], W0))

    acc = lax.fori_loop(0, N // BLK, blk, jnp.ones((8, RANK), jnp.float32))
    p0_ref[...] = _rows8_prod(acc)


def _tc_mid_body(x0_ref, x1_ref, a0_ref, a1_ref, W_ref,
                 h0_ref, h1_ref, p1_ref):
    W1 = W_ref[1]

    def blk(i, acc):
        sl = pl.ds(i * BLK, BLK)
        h0 = x0_ref[sl, :] + a0_ref[0, sl, :] + a0_ref[1, sl, :]
        h1 = x1_ref[sl, :] + a1_ref[0, sl, :] + a1_ref[1, sl, :]
        h0_ref[sl, :] = h0
        h1_ref[sl, :] = h1
        return acc * _block_prod(_tanh_z(h0, h1, W1))

    acc = lax.fori_loop(0, N // BLK, blk, jnp.ones((8, RANK), jnp.float32))
    p1_ref[...] = _rows8_prod(acc)


def _tc_final_body(h0_ref, h1_ref, a0_ref, a1_ref, W_ref, p0_ref, p1_ref,
                   lw_ref, lb_ref, score_ref):
    W2 = W_ref[2]

    def blk(i, acc):
        sl = pl.ds(i * BLK, BLK)
        g0 = h0_ref[sl, :] + a0_ref[0, sl, :] + a0_ref[1, sl, :]
        g1 = h1_ref[sl, :] + a1_ref[0, sl, :] + a1_ref[1, sl, :]
        return acc * _block_prod(_tanh_z(g0, g1, W2))

    acc = lax.fori_loop(0, N // BLK, blk, jnp.ones((8, RANK), jnp.float32))
    p2 = _rows8_prod(acc)
    pools = (p0_ref[...], p1_ref[...], p2)
    score = jnp.zeros((1, ODIM), jnp.float32)
    for l in range(3):
        score = score + lax.dot_general(
            pools[l], lw_ref[l], (((1,), (1,)), ((), ())),
            precision=lax.Precision.HIGHEST) + lb_ref[l:l + 1, :]
    score_ref[...] = score


_tc_pool0 = pl.pallas_call(
    _tc_pool0_body,
    out_shape=jax.ShapeDtypeStruct((1, RANK), jnp.float32),
)

_tc_mid = pl.pallas_call(
    _tc_mid_body,
    out_shape=[
        jax.ShapeDtypeStruct((N, HD), jnp.float32),
        jax.ShapeDtypeStruct((N, HD), jnp.float32),
        jax.ShapeDtypeStruct((1, RANK), jnp.float32),
    ],
)

_tc_final = pl.pallas_call(
    _tc_final_body,
    out_shape=jax.ShapeDtypeStruct((1, ODIM), jnp.float32),
)


def kernel(x, edge_index, W_pools, lin_w, lin_b):
    row = edge_index[0]
    col = edge_index[1]
    pad = EPAD - E
    row_p = jnp.concatenate(
        [row, jnp.full((pad,), N, jnp.int32)]).reshape(NW, CHUNKS, CHUNK)
    col_p = jnp.concatenate(
        [col, jnp.zeros((pad,), jnp.int32)]).reshape(NW, CHUNKS, CHUNK)
    zeros = jnp.zeros((CHUNK, HD), jnp.float32)
    x0 = x[:, 0:HD]
    x1 = x[:, HD:D]

    a0, a1 = _sc_agg(x0, x1, col_p, row_p, zeros)
    p0 = _tc_pool0(x0, x1, W_pools)
    h0, h1, p1 = _tc_mid(x0, x1, a0, a1, W_pools)
    b0, b1 = _sc_agg(h0, h1, col_p, row_p, zeros)
    return _tc_final(h0, h1, b0, b1, W_pools, p0, p1, lin_w, lin_b)
